# baseline (speedup 1.0000x reference)
import functools

import jax
import jax.numpy as jnp
from jax import lax
from jax.experimental import pallas as pl
from jax.experimental.pallas import tpu as pltpu
from jax.experimental.pallas import tpu_sc as plsc

_L = 16


def _rsqrt(x):
    xi = plsc.bitcast(x, jnp.int32)
    y = plsc.bitcast(0x5F3759DF - (xi >> 1), jnp.float32)
    for _ in range(3):
        y = y * (1.5 - 0.5 * x * y * y)
    return y


def _build(P, D, NC, NS):
    NW = NC * NS
    BPW = P // NW
    NG = BPW // _L
    DC = D // _L

    mesh = plsc.VectorSubcoreMesh(core_axis_name="c", subcore_axis_name="s")

    @functools.partial(
        pl.kernel,
        mesh=mesh,
        out_type=jax.ShapeDtypeStruct((NW, _L), jnp.float32),
        compiler_params=pltpu.CompilerParams(
            use_tc_tiling_on_sc=False, needs_layout_passes=False),
        scratch_types=[
            pltpu.VMEM((BPW,), jnp.int32),
            pltpu.VMEM((BPW,), jnp.int32),
            pltpu.VMEM((BPW,), jnp.int32),
            pltpu.VMEM((BPW, D), jnp.float32),
            pltpu.VMEM((BPW, D), jnp.float32),
            pltpu.VMEM((NG, _L), jnp.float32),
            pltpu.VMEM((NG, _L), jnp.float32),
            pltpu.VMEM((NG, _L), jnp.float32),
            pltpu.VMEM((_L,), jnp.float32),
            pltpu.SemaphoreType.DMA,
        ],
    )
    def sc_loss(emb_hbm, kg_hbm, ii_hbm, jj_hbm, nn_hbm, out_hbm,
                idx_i, idx_j, idx_n, buf_a, buf_b,
                sim_eb, sq_a, neg_c, acc_v, sem):
        wid = lax.axis_index("s") * NC + lax.axis_index("c")
        base = wid * BPW
        pltpu.sync_copy(ii_hbm.at[pl.ds(base, BPW)], idx_i)
        pltpu.sync_copy(jj_hbm.at[pl.ds(base, BPW)], idx_j)
        pltpu.sync_copy(nn_hbm.at[pl.ds(base, BPW)], idx_n)

        def dots3(g):
            rows = lax.iota(jnp.int32, _L) + g * _L

            def d_body(c, carry):
                ab, aa, bb = carry
                for u in range(_L):
                    col = jnp.full((_L,), c * _L + u, dtype=jnp.int32)
                    a = plsc.load_gather(buf_a, [rows, col])
                    b = plsc.load_gather(buf_b, [rows, col])
                    ab = ab + a * b
                    aa = aa + a * a
                    bb = bb + b * b
                return ab, aa, bb

            z = jnp.zeros((_L,), jnp.float32)
            return lax.fori_loop(0, DC, d_body, (z, z, z))

        def cos(ab, aa, bb):
            nn2 = aa * bb
            nrm = nn2 * _rsqrt(nn2)
            return ab / jnp.maximum(nrm, 1e-8)

        pltpu.async_copy(emb_hbm.at[idx_i], buf_a, sem).wait()
        pltpu.async_copy(emb_hbm.at[idx_j], buf_b, sem).wait()

        def g1(g, _):
            ab, aa, bb = dots3(g)
            sim_eb[g, :] = cos(ab, aa, bb)
            sq_a[g, :] = aa
            return 0
        lax.fori_loop(0, NG, g1, 0)

        pltpu.async_copy(emb_hbm.at[idx_n], buf_b, sem).wait()

        def g2(g, _):
            ab, _aa, bb = dots3(g)
            neg_sim = cos(ab, sq_a[g, :], bb)
            neg_c[g, :] = 0.1 * jnp.maximum(0.0, 0.5 - neg_sim)
            return 0
        lax.fori_loop(0, NG, g2, 0)

        pltpu.async_copy(kg_hbm.at[idx_i], buf_a, sem).wait()
        pltpu.async_copy(kg_hbm.at[idx_j], buf_b, sem).wait()

        def g3(g, acc):
            ab, aa, bb = dots3(g)
            sim_kg = cos(ab, aa, bb)
            per = jnp.abs(sim_eb[g, :] - sim_kg) + neg_c[g, :]
            iv = idx_i[pl.ds(g * _L, _L)]
            jv = idx_j[pl.ds(g * _L, _L)]
            nv = idx_n[pl.ds(g * _L, _L)]
            mask = (nv != iv) & (nv != jv)
            return acc + jnp.where(mask, per, 0.0)
        acc = lax.fori_loop(0, NG, g3, jnp.zeros((_L,), jnp.float32))

        acc_v[...] = acc
        pltpu.sync_copy(acc_v, out_hbm.at[wid])

    return sc_loss


def kernel(entity_embeddings, knowledge_embeddings, entity_pairs):
    P = entity_pairs.shape[0]
    N, D = entity_embeddings.shape
    info = plsc.get_sparse_core_info()
    NC, NS = info.num_cores, info.num_subcores

    ii = entity_pairs[:, 0].astype(jnp.int32)
    jj = entity_pairs[:, 1].astype(jnp.int32)
    nn = jax.random.randint(jax.random.key(42), (P,), 0, N).astype(jnp.int32)

    partials = _build(P, D, NC, NS)(
        entity_embeddings, knowledge_embeddings, ii, jj, nn)
    return jnp.sum(partials) / max(P, 1)


# speedup vs baseline: 2.3700x; 2.3700x over previous
import functools

import jax
import jax.numpy as jnp
from jax import lax
from jax.experimental import pallas as pl
from jax.experimental.pallas import tpu as pltpu
from jax.experimental.pallas import tpu_sc as plsc

_L = 16


def _rsqrt(x):
    xi = plsc.bitcast(x, jnp.int32)
    y = plsc.bitcast(0x5F3759DF - (xi >> 1), jnp.float32)
    for _ in range(3):
        y = y * (1.5 - 0.5 * x * y * y)
    return y


def _build(P, D, NC, NS):
    NW = NC * NS
    BPW = P // NW
    NG = BPW // _L
    UP = 8

    mesh = plsc.VectorSubcoreMesh(core_axis_name="c", subcore_axis_name="s")

    @functools.partial(
        pl.kernel,
        mesh=mesh,
        out_type=jax.ShapeDtypeStruct((NW, _L), jnp.float32),
        compiler_params=pltpu.CompilerParams(
            use_tc_tiling_on_sc=False, needs_layout_passes=False),
        scratch_types=[
            pltpu.VMEM((BPW,), jnp.int32),
            pltpu.VMEM((BPW,), jnp.int32),
            pltpu.VMEM((BPW,), jnp.int32),
            pltpu.VMEM((BPW, D), jnp.float32),
            pltpu.VMEM((BPW, D), jnp.float32),
            pltpu.VMEM((BPW, D), jnp.float32),
            pltpu.VMEM((8, BPW), jnp.float32),
            pltpu.VMEM((_L,), jnp.float32),
            pltpu.SemaphoreType.DMA,
            pltpu.SemaphoreType.DMA,
            pltpu.SemaphoreType.DMA,
        ],
    )
    def sc_loss(emb_hbm, kg_hbm, ii_hbm, jj_hbm, nn_hbm, out_hbm,
                idx_i, idx_j, idx_n, buf_a, buf_b, buf_c,
                dots, acc_v, sem_a, sem_b, sem_c):
        wid = lax.axis_index("s") * NC + lax.axis_index("c")
        base = wid * BPW
        pltpu.sync_copy(ii_hbm.at[pl.ds(base, BPW)], idx_i)
        pltpu.sync_copy(jj_hbm.at[pl.ds(base, BPW)], idx_j)
        pltpu.sync_copy(nn_hbm.at[pl.ds(base, BPW)], idx_n)

        cp_a = pltpu.async_copy(emb_hbm.at[idx_i], buf_a, sem_a)
        cp_b = pltpu.async_copy(emb_hbm.at[idx_j], buf_b, sem_b)
        cp_c = pltpu.async_copy(emb_hbm.at[idx_n], buf_c, sem_c)

        def dot_pass(x_ref, y_ref, slots):
            def g_loop(g, _):
                rows = lax.iota(jnp.int32, _L) + g * _L

                def d_body(c, carry):
                    colv, xy, xx, yy = carry
                    for _u in range(UP):
                        x = plsc.load_gather(x_ref, [rows, colv])
                        y = plsc.load_gather(y_ref, [rows, colv])
                        xy = xy + x * y
                        xx = xx + x * x
                        yy = yy + y * y
                        colv = (colv + 1) & (D - 1)
                    return colv, xy, xx, yy

                z = jnp.zeros((_L,), jnp.float32)
                _, xy, xx, yy = lax.fori_loop(
                    0, D // UP, d_body, (lax.iota(jnp.int32, _L), z, z, z))
                vals = {'xy': xy, 'xx': xx, 'yy': yy}
                s = pl.ds(g * _L, _L)
                for row, kind in slots:
                    dots[row, s] = vals[kind]
                return 0
            lax.fori_loop(0, NG, g_loop, 0)

        cp_a.wait()
        cp_b.wait()
        dot_pass(buf_a, buf_b, [(0, 'xy'), (1, 'xx'), (2, 'yy')])

        cp_b2 = pltpu.async_copy(kg_hbm.at[idx_i], buf_b, sem_b)
        cp_c.wait()
        dot_pass(buf_a, buf_c, [(3, 'xy'), (4, 'yy')])

        cp_a2 = pltpu.async_copy(kg_hbm.at[idx_j], buf_a, sem_a)
        cp_b2.wait()
        cp_a2.wait()
        dot_pass(buf_b, buf_a, [(5, 'xy'), (6, 'xx'), (7, 'yy')])

        def cos(ab, aa, bb):
            n2 = aa * bb
            nrm = n2 * _rsqrt(n2)
            return ab / jnp.maximum(nrm, 1e-8)

        def g_body(g, acc):
            s = pl.ds(g * _L, _L)
            sim_eb = cos(dots[0, s], dots[1, s], dots[2, s])
            neg_sim = cos(dots[3, s], dots[1, s], dots[4, s])
            sim_kg = cos(dots[5, s], dots[6, s], dots[7, s])
            per = (jnp.abs(sim_eb - sim_kg)
                   + 0.1 * jnp.maximum(0.0, 0.5 - neg_sim))
            iv, jv, nv = idx_i[s], idx_j[s], idx_n[s]
            mask = (nv != iv) & (nv != jv)
            return acc + jnp.where(mask, per, 0.0)
        acc = lax.fori_loop(0, NG, g_body, jnp.zeros((_L,), jnp.float32))

        acc_v[...] = acc
        pltpu.sync_copy(acc_v, out_hbm.at[wid])

    return sc_loss


def kernel(entity_embeddings, knowledge_embeddings, entity_pairs):
    P = entity_pairs.shape[0]
    N, D = entity_embeddings.shape
    info = plsc.get_sparse_core_info()
    NC, NS = info.num_cores, info.num_subcores

    ii = entity_pairs[:, 0].astype(jnp.int32)
    jj = entity_pairs[:, 1].astype(jnp.int32)
    nn = jax.random.randint(jax.random.key(42), (P,), 0, N).astype(jnp.int32)

    partials = _build(P, D, NC, NS)(
        entity_embeddings, knowledge_embeddings, ii, jj, nn)
    return jnp.sum(partials) / max(P, 1)


# speedup vs baseline: 3.0665x; 1.2939x over previous
import functools

import jax
import jax.numpy as jnp
from jax import lax
from jax.experimental import pallas as pl
from jax.experimental.pallas import tpu as pltpu
from jax.experimental.pallas import tpu_sc as plsc

_L = 16


def _rsqrt(x):
    xi = plsc.bitcast(x, jnp.int32)
    y = plsc.bitcast(0x5F3759DF - (xi >> 1), jnp.float32)
    for _ in range(3):
        y = y * (1.5 - 0.5 * x * y * y)
    return y


def _build(P, D, NC, NS):
    NW = NC * NS
    BPW = P // NW
    NG = BPW // _L
    UP = 8

    mesh = plsc.VectorSubcoreMesh(core_axis_name="c", subcore_axis_name="s")

    @functools.partial(
        pl.kernel,
        mesh=mesh,
        out_type=jax.ShapeDtypeStruct((NW, _L), jnp.float32),
        compiler_params=pltpu.CompilerParams(
            use_tc_tiling_on_sc=True, needs_layout_passes=False),
        scratch_types=[
            pltpu.VMEM((BPW,), jnp.int32),
            pltpu.VMEM((BPW,), jnp.int32),
            pltpu.VMEM((BPW,), jnp.int32),
            pltpu.VMEM((BPW, D), jnp.float32),
            pltpu.VMEM((BPW, D), jnp.float32),
            pltpu.VMEM((BPW, D), jnp.float32),
            pltpu.VMEM((8, BPW), jnp.float32),
            pltpu.VMEM((_L,), jnp.float32),
            pltpu.SemaphoreType.DMA,
            pltpu.SemaphoreType.DMA,
            pltpu.SemaphoreType.DMA,
        ],
    )
    def sc_loss(emb_hbm, kg_hbm, ii_hbm, jj_hbm, nn_hbm, out_hbm,
                idx_i, idx_j, idx_n, buf_a, buf_b, buf_c,
                dots, acc_v, sem_a, sem_b, sem_c):
        wid = lax.axis_index("s") * NC + lax.axis_index("c")
        base = wid * BPW
        pltpu.sync_copy(ii_hbm.at[pl.ds(base, BPW)], idx_i)
        pltpu.sync_copy(jj_hbm.at[pl.ds(base, BPW)], idx_j)
        pltpu.sync_copy(nn_hbm.at[pl.ds(base, BPW)], idx_n)

        cp_a = pltpu.async_copy(emb_hbm.at[idx_i], buf_a, sem_a)
        cp_b = pltpu.async_copy(emb_hbm.at[idx_j], buf_b, sem_b)
        cp_c = pltpu.async_copy(emb_hbm.at[idx_n], buf_c, sem_c)

        def dot_pass(x_ref, y_ref, slots):
            def g_loop(g, _):
                rows = lax.iota(jnp.int32, _L) + g * _L

                def d_body(c, carry):
                    colv, xy, xx, yy = carry
                    for _u in range(UP):
                        x = plsc.load_gather(x_ref, [rows, colv])
                        y = plsc.load_gather(y_ref, [rows, colv])
                        xy = xy + x * y
                        xx = xx + x * x
                        yy = yy + y * y
                        colv = (colv + 1) & (D - 1)
                    return colv, xy, xx, yy

                z = jnp.zeros((_L,), jnp.float32)
                _, xy, xx, yy = lax.fori_loop(
                    0, D // UP, d_body, (lax.iota(jnp.int32, _L), z, z, z))
                vals = {'xy': xy, 'xx': xx, 'yy': yy}
                s = pl.ds(g * _L, _L)
                for row, kind in slots:
                    dots[row, s] = vals[kind]
                return 0
            lax.fori_loop(0, NG, g_loop, 0)

        cp_a.wait()
        cp_b.wait()
        dot_pass(buf_a, buf_b, [(0, 'xy'), (1, 'xx'), (2, 'yy')])

        cp_b2 = pltpu.async_copy(kg_hbm.at[idx_i], buf_b, sem_b)
        cp_c.wait()
        dot_pass(buf_a, buf_c, [(3, 'xy'), (4, 'yy')])

        cp_a2 = pltpu.async_copy(kg_hbm.at[idx_j], buf_a, sem_a)
        cp_b2.wait()
        cp_a2.wait()
        dot_pass(buf_b, buf_a, [(5, 'xy'), (6, 'xx'), (7, 'yy')])

        def cos(ab, aa, bb):
            n2 = aa * bb
            nrm = n2 * _rsqrt(n2)
            return ab / jnp.maximum(nrm, 1e-8)

        def g_body(g, acc):
            s = pl.ds(g * _L, _L)
            sim_eb = cos(dots[0, s], dots[1, s], dots[2, s])
            neg_sim = cos(dots[3, s], dots[1, s], dots[4, s])
            sim_kg = cos(dots[5, s], dots[6, s], dots[7, s])
            per = (jnp.abs(sim_eb - sim_kg)
                   + 0.1 * jnp.maximum(0.0, 0.5 - neg_sim))
            iv, jv, nv = idx_i[s], idx_j[s], idx_n[s]
            mask = (nv != iv) & (nv != jv)
            return acc + jnp.where(mask, per, 0.0)
        acc = lax.fori_loop(0, NG, g_body, jnp.zeros((_L,), jnp.float32))

        acc_v[...] = acc
        pltpu.sync_copy(acc_v, out_hbm.at[wid])

    return sc_loss


def kernel(entity_embeddings, knowledge_embeddings, entity_pairs):
    P = entity_pairs.shape[0]
    N, D = entity_embeddings.shape
    info = plsc.get_sparse_core_info()
    NC, NS = info.num_cores, info.num_subcores

    ii = entity_pairs[:, 0].astype(jnp.int32)
    jj = entity_pairs[:, 1].astype(jnp.int32)
    nn = jax.random.randint(jax.random.key(42), (P,), 0, N).astype(jnp.int32)

    partials = _build(P, D, NC, NS)(
        entity_embeddings, knowledge_embeddings, ii, jj, nn)
    return jnp.sum(partials) / max(P, 1)


# speedup vs baseline: 3.6787x; 1.1996x over previous
import functools

import jax
import jax.numpy as jnp
import numpy as np
from jax import lax
from jax.experimental import pallas as pl
from jax.experimental.pallas import tpu as pltpu
from jax.experimental.pallas import tpu_sc as plsc

_L = 16


def _np_threefry2x32(k0, k1, x0, x1):
    rot0 = (13, 15, 26, 6)
    rot1 = (17, 29, 16, 24)
    k0 = np.uint32(k0)
    k1 = np.uint32(k1)
    ks = (k0, k1, np.uint32(k0 ^ k1 ^ np.uint32(0x1BD11BDA)))
    x0 = x0.astype(np.uint32)
    x1 = x1.astype(np.uint32)

    def rotl(v, r):
        return ((v << np.uint32(r)) | (v >> np.uint32(32 - r))).astype(np.uint32)

    with np.errstate(over='ignore'):
        x0 = x0 + ks[0]
        x1 = x1 + ks[1]
        for i in range(5):
            for r in (rot0 if i % 2 == 0 else rot1):
                x0 = x0 + x1
                x1 = rotl(x1, r)
                x1 = x1 ^ x0
            x0 = x0 + ks[(i + 1) % 3]
            x1 = x1 + ks[(i + 2) % 3] + np.uint32(i + 1)
    return x0, x1


def _np_random_bits(k0, k1, n):
    o0, o1 = _np_threefry2x32(k0, k1, np.zeros(n, np.uint32),
                              np.arange(n, dtype=np.uint32))
    return (o0 ^ o1).astype(np.uint32)


@functools.lru_cache(maxsize=None)
def _neg_indices(seed, n, span):
    k0 = np.uint32((seed >> 32) & 0xFFFFFFFF)
    k1 = np.uint32(seed & 0xFFFFFFFF)
    o0, o1 = _np_threefry2x32(k0, k1, np.array([0, 0], np.uint32),
                              np.array([0, 1], np.uint32))
    higher = _np_random_bits(o0[0], o1[0], n)
    lower = _np_random_bits(o0[1], o1[1], n)
    span_u = np.uint32(span)
    with np.errstate(over='ignore'):
        multiplier = np.uint32(np.uint32(1 << 16) % span_u)
        multiplier = np.uint32((multiplier * multiplier) % span_u)
        off = ((higher % span_u) * multiplier + (lower % span_u)) % span_u
    return np.asarray(off, np.int32)


def _rsqrt(x):
    xi = plsc.bitcast(x, jnp.int32)
    y = plsc.bitcast(0x5F3759DF - (xi >> 1), jnp.float32)
    for _ in range(3):
        y = y * (1.5 - 0.5 * x * y * y)
    return y


def _build(P, D, NC, NS):
    NW = NC * NS
    BPW = P // NW
    NG = BPW // _L
    DH = D // 2
    UP = 8

    mesh = plsc.VectorSubcoreMesh(core_axis_name="c", subcore_axis_name="s")

    @functools.partial(
        pl.kernel,
        mesh=mesh,
        out_type=jax.ShapeDtypeStruct((NW, _L), jnp.float32),
        compiler_params=pltpu.CompilerParams(
            use_tc_tiling_on_sc=True, needs_layout_passes=False),
        scratch_types=[
            pltpu.VMEM((BPW,), jnp.int32),
            pltpu.VMEM((BPW,), jnp.int32),
            pltpu.VMEM((BPW,), jnp.int32),
            pltpu.VMEM((BPW, D), jnp.float32),
            pltpu.VMEM((BPW, D), jnp.float32),
            pltpu.VMEM((BPW, D), jnp.float32),
            pltpu.VMEM((8, BPW), jnp.float32),
            pltpu.VMEM((_L,), jnp.float32),
            pltpu.SemaphoreType.DMA,
            pltpu.SemaphoreType.DMA,
            pltpu.SemaphoreType.DMA,
        ],
    )
    def sc_loss(emb_hbm, kg_hbm, ii_hbm, jj_hbm, nn_hbm, out_hbm,
                idx_i, idx_j, idx_n, buf_a, buf_b, buf_c,
                dots, acc_v, sem_a, sem_b, sem_c):
        wid = lax.axis_index("s") * NC + lax.axis_index("c")
        base = wid * BPW
        pltpu.sync_copy(ii_hbm.at[pl.ds(base, BPW)], idx_i)
        pltpu.sync_copy(jj_hbm.at[pl.ds(base, BPW)], idx_j)
        pltpu.sync_copy(nn_hbm.at[pl.ds(base, BPW)], idx_n)

        cp_a = pltpu.async_copy(emb_hbm.at[idx_i], buf_a, sem_a)
        cp_b = pltpu.async_copy(emb_hbm.at[idx_j], buf_b, sem_b)
        cp_c = pltpu.async_copy(emb_hbm.at[idx_n], buf_c, sem_c)

        def dot_pass(x_ref, y_ref, slots):
            def g_loop(g, _):
                rows = lax.iota(jnp.int32, _L) + g * _L
                z = jnp.zeros((_L,), jnp.float32)
                acc = (z, z, z)
                for h in (0, DH):
                    def d_body(c, carry):
                        colv, xy, xx, yy = carry
                        for _u in range(UP):
                            cols = colv + h
                            x = plsc.load_gather(x_ref, [rows, cols])
                            y = plsc.load_gather(y_ref, [rows, cols])
                            xy = xy + x * y
                            xx = xx + x * x
                            yy = yy + y * y
                            colv = (colv + 1) & (DH - 1)
                        return colv, xy, xx, yy

                    _, *acc = lax.fori_loop(
                        0, DH // UP, d_body,
                        (lax.iota(jnp.int32, _L), *acc))
                xy, xx, yy = acc
                vals = {'xy': xy, 'xx': xx, 'yy': yy}
                s = pl.ds(g * _L, _L)
                for row, kind in slots:
                    dots[row, s] = vals[kind]
                return 0
            lax.fori_loop(0, NG, g_loop, 0)

        cp_a.wait()
        cp_b.wait()
        dot_pass(buf_a, buf_b, [(0, 'xy'), (1, 'xx'), (2, 'yy')])

        cp_b2 = pltpu.async_copy(kg_hbm.at[idx_i], buf_b, sem_b)
        cp_c.wait()
        dot_pass(buf_a, buf_c, [(3, 'xy'), (4, 'yy')])

        cp_a2 = pltpu.async_copy(kg_hbm.at[idx_j], buf_a, sem_a)
        cp_b2.wait()
        cp_a2.wait()
        dot_pass(buf_b, buf_a, [(5, 'xy'), (6, 'xx'), (7, 'yy')])

        def cos(ab, aa, bb):
            n2 = aa * bb
            nrm = n2 * _rsqrt(n2)
            return ab / jnp.maximum(nrm, 1e-8)

        def g_body(g, acc):
            s = pl.ds(g * _L, _L)
            sim_eb = cos(dots[0, s], dots[1, s], dots[2, s])
            neg_sim = cos(dots[3, s], dots[1, s], dots[4, s])
            sim_kg = cos(dots[5, s], dots[6, s], dots[7, s])
            per = (jnp.abs(sim_eb - sim_kg)
                   + 0.1 * jnp.maximum(0.0, 0.5 - neg_sim))
            iv, jv, nv = idx_i[s], idx_j[s], idx_n[s]
            mask = (nv != iv) & (nv != jv)
            return acc + jnp.where(mask, per, 0.0)
        acc = lax.fori_loop(0, NG, g_body, jnp.zeros((_L,), jnp.float32))

        acc_v[...] = acc
        pltpu.sync_copy(acc_v, out_hbm.at[wid])

    return sc_loss


def kernel(entity_embeddings, knowledge_embeddings, entity_pairs):
    P = entity_pairs.shape[0]
    N, D = entity_embeddings.shape
    info = plsc.get_sparse_core_info()
    NC, NS = info.num_cores, info.num_subcores

    ii = entity_pairs[:, 0].astype(jnp.int32)
    jj = entity_pairs[:, 1].astype(jnp.int32)
    nn = jnp.asarray(_neg_indices(42, P, N))

    partials = _build(P, D, NC, NS)(
        entity_embeddings, knowledge_embeddings, ii, jj, nn)
    return jnp.sum(partials) / max(P, 1)


# speedup vs baseline: 3.8701x; 1.0520x over previous
import functools

import jax
import jax.numpy as jnp
import numpy as np
from jax import lax
from jax.experimental import pallas as pl
from jax.experimental.pallas import tpu as pltpu
from jax.experimental.pallas import tpu_sc as plsc

_L = 16


def _np_threefry2x32(k0, k1, x0, x1):
    rot0 = (13, 15, 26, 6)
    rot1 = (17, 29, 16, 24)
    k0 = np.uint32(k0)
    k1 = np.uint32(k1)
    ks = (k0, k1, np.uint32(k0 ^ k1 ^ np.uint32(0x1BD11BDA)))
    x0 = x0.astype(np.uint32)
    x1 = x1.astype(np.uint32)

    def rotl(v, r):
        return ((v << np.uint32(r)) | (v >> np.uint32(32 - r))).astype(np.uint32)

    with np.errstate(over='ignore'):
        x0 = x0 + ks[0]
        x1 = x1 + ks[1]
        for i in range(5):
            for r in (rot0 if i % 2 == 0 else rot1):
                x0 = x0 + x1
                x1 = rotl(x1, r)
                x1 = x1 ^ x0
            x0 = x0 + ks[(i + 1) % 3]
            x1 = x1 + ks[(i + 2) % 3] + np.uint32(i + 1)
    return x0, x1


def _np_random_bits(k0, k1, n):
    o0, o1 = _np_threefry2x32(k0, k1, np.zeros(n, np.uint32),
                              np.arange(n, dtype=np.uint32))
    return (o0 ^ o1).astype(np.uint32)


@functools.lru_cache(maxsize=None)
def _neg_indices(seed, n, span):
    k0 = np.uint32((seed >> 32) & 0xFFFFFFFF)
    k1 = np.uint32(seed & 0xFFFFFFFF)
    o0, o1 = _np_threefry2x32(k0, k1, np.array([0, 0], np.uint32),
                              np.array([0, 1], np.uint32))
    higher = _np_random_bits(o0[0], o1[0], n)
    lower = _np_random_bits(o0[1], o1[1], n)
    span_u = np.uint32(span)
    with np.errstate(over='ignore'):
        multiplier = np.uint32(np.uint32(1 << 16) % span_u)
        multiplier = np.uint32((multiplier * multiplier) % span_u)
        off = ((higher % span_u) * multiplier + (lower % span_u)) % span_u
    return np.asarray(off, np.int32)


def _rsqrt(x):
    xi = plsc.bitcast(x, jnp.int32)
    y = plsc.bitcast(0x5F3759DF - (xi >> 1), jnp.float32)
    for _ in range(3):
        y = y * (1.5 - 0.5 * x * y * y)
    return y


def _build(P, D, NC, NS):
    NW = NC * NS
    BPW = P // NW
    NG = BPW // _L
    DH = D // 2
    UP = 8

    mesh = plsc.VectorSubcoreMesh(core_axis_name="c", subcore_axis_name="s")

    @functools.partial(
        pl.kernel,
        mesh=mesh,
        out_type=jax.ShapeDtypeStruct((NW, _L), jnp.float32),
        compiler_params=pltpu.CompilerParams(
            use_tc_tiling_on_sc=True, needs_layout_passes=False),
        scratch_types=[
            pltpu.VMEM((BPW,), jnp.int32),
            pltpu.VMEM((BPW,), jnp.int32),
            pltpu.VMEM((BPW,), jnp.int32),
            pltpu.VMEM((BPW, D), jnp.float32),
            pltpu.VMEM((BPW, D), jnp.float32),
            pltpu.VMEM((BPW, D), jnp.float32),
            pltpu.VMEM((8, BPW), jnp.float32),
            pltpu.VMEM((_L,), jnp.float32),
            pltpu.SemaphoreType.DMA,
            pltpu.SemaphoreType.DMA,
            pltpu.SemaphoreType.DMA,
        ],
    )
    def sc_loss(emb_hbm, kg_hbm, ii_hbm, jj_hbm, nn_hbm, out_hbm,
                idx_i, idx_j, idx_n, buf_a, buf_b, buf_c,
                dots, acc_v, sem_a, sem_b, sem_c):
        wid = lax.axis_index("s") * NC + lax.axis_index("c")
        base = wid * BPW
        ci = pltpu.async_copy(ii_hbm.at[pl.ds(base, BPW)], idx_i, sem_a)
        cj = pltpu.async_copy(jj_hbm.at[pl.ds(base, BPW)], idx_j, sem_b)
        cn = pltpu.async_copy(nn_hbm.at[pl.ds(base, BPW)], idx_n, sem_c)
        ci.wait()
        cj.wait()
        cn.wait()

        HB = BPW // 2
        hs0, hs1 = pl.ds(0, HB), pl.ds(HB, HB)
        cp_a1 = pltpu.async_copy(emb_hbm.at[idx_i.at[hs0]],
                                 buf_a.at[hs0, :], sem_a)
        cp_b1 = pltpu.async_copy(emb_hbm.at[idx_j.at[hs0]],
                                 buf_b.at[hs0, :], sem_b)
        cp_a2 = pltpu.async_copy(emb_hbm.at[idx_i.at[hs1]],
                                 buf_a.at[hs1, :], sem_a)
        cp_b2 = pltpu.async_copy(emb_hbm.at[idx_j.at[hs1]],
                                 buf_b.at[hs1, :], sem_b)
        cp_c = pltpu.async_copy(emb_hbm.at[idx_n], buf_c, sem_c)

        def dot_pass(x_ref, y_ref, slots, g_lo, g_hi):
            def g_loop(g, _):
                rows = lax.iota(jnp.int32, _L) + g * _L
                z = jnp.zeros((_L,), jnp.float32)
                acc = (z, z, z, z, z, z)
                for h in (0, DH):
                    def d_body(c, carry):
                        colv, xy0, xy1, xx0, xx1, yy0, yy1 = carry
                        for u in range(UP):
                            cols = colv + h
                            x = plsc.load_gather(x_ref, [rows, cols])
                            y = plsc.load_gather(y_ref, [rows, cols])
                            if u % 2 == 0:
                                xy0 = xy0 + x * y
                                xx0 = xx0 + x * x
                                yy0 = yy0 + y * y
                            else:
                                xy1 = xy1 + x * y
                                xx1 = xx1 + x * x
                                yy1 = yy1 + y * y
                            colv = (colv + 1) & (DH - 1)
                        return colv, xy0, xy1, xx0, xx1, yy0, yy1

                    _, *acc = lax.fori_loop(
                        0, DH // UP, d_body,
                        (lax.iota(jnp.int32, _L), *acc))
                xy0, xy1, xx0, xx1, yy0, yy1 = acc
                vals = {'xy': xy0 + xy1, 'xx': xx0 + xx1, 'yy': yy0 + yy1}
                s = pl.ds(g * _L, _L)
                for row, kind in slots:
                    dots[row, s] = vals[kind]
                return 0
            lax.fori_loop(g_lo, g_hi, g_loop, 0)

        s1 = [(0, 'xy'), (1, 'xx'), (2, 'yy')]
        cp_a1.wait()
        cp_b1.wait()
        dot_pass(buf_a, buf_b, s1, 0, NG // 2)
        cp_a2.wait()
        cp_b2.wait()
        dot_pass(buf_a, buf_b, s1, NG // 2, NG)

        cp_bk = pltpu.async_copy(kg_hbm.at[idx_i], buf_b, sem_b)
        cp_c.wait()
        dot_pass(buf_a, buf_c, [(3, 'xy'), (4, 'yy')], 0, NG)

        cp_ak = pltpu.async_copy(kg_hbm.at[idx_j], buf_a, sem_a)
        cp_bk.wait()
        cp_ak.wait()
        dot_pass(buf_b, buf_a, [(5, 'xy'), (6, 'xx'), (7, 'yy')], 0, NG)

        def cos(ab, aa, bb):
            n2 = aa * bb
            nrm = n2 * _rsqrt(n2)
            return ab / jnp.maximum(nrm, 1e-8)

        def g_body(g, acc):
            s = pl.ds(g * _L, _L)
            sim_eb = cos(dots[0, s], dots[1, s], dots[2, s])
            neg_sim = cos(dots[3, s], dots[1, s], dots[4, s])
            sim_kg = cos(dots[5, s], dots[6, s], dots[7, s])
            per = (jnp.abs(sim_eb - sim_kg)
                   + 0.1 * jnp.maximum(0.0, 0.5 - neg_sim))
            iv, jv, nv = idx_i[s], idx_j[s], idx_n[s]
            mask = (nv != iv) & (nv != jv)
            return acc + jnp.where(mask, per, 0.0)
        acc = lax.fori_loop(0, NG, g_body, jnp.zeros((_L,), jnp.float32))

        acc_v[...] = acc
        pltpu.sync_copy(acc_v, out_hbm.at[wid])

    return sc_loss


def kernel(entity_embeddings, knowledge_embeddings, entity_pairs):
    P = entity_pairs.shape[0]
    N, D = entity_embeddings.shape
    info = plsc.get_sparse_core_info()
    NC, NS = info.num_cores, info.num_subcores

    ii = entity_pairs[:, 0].astype(jnp.int32)
    jj = entity_pairs[:, 1].astype(jnp.int32)
    nn = jnp.asarray(_neg_indices(42, P, N))

    partials = _build(P, D, NC, NS)(
        entity_embeddings, knowledge_embeddings, ii, jj, nn)
    return jnp.sum(partials) / max(P, 1)


# speedup vs baseline: 4.1081x; 1.0615x over previous
import functools

import jax
import jax.numpy as jnp
import numpy as np
from jax import lax
from jax.experimental import pallas as pl
from jax.experimental.pallas import tpu as pltpu
from jax.experimental.pallas import tpu_sc as plsc

_L = 16


def _np_threefry2x32(k0, k1, x0, x1):
    rot0 = (13, 15, 26, 6)
    rot1 = (17, 29, 16, 24)
    k0 = np.uint32(k0)
    k1 = np.uint32(k1)
    ks = (k0, k1, np.uint32(k0 ^ k1 ^ np.uint32(0x1BD11BDA)))
    x0 = x0.astype(np.uint32)
    x1 = x1.astype(np.uint32)

    def rotl(v, r):
        return ((v << np.uint32(r)) | (v >> np.uint32(32 - r))).astype(np.uint32)

    with np.errstate(over='ignore'):
        x0 = x0 + ks[0]
        x1 = x1 + ks[1]
        for i in range(5):
            for r in (rot0 if i % 2 == 0 else rot1):
                x0 = x0 + x1
                x1 = rotl(x1, r)
                x1 = x1 ^ x0
            x0 = x0 + ks[(i + 1) % 3]
            x1 = x1 + ks[(i + 2) % 3] + np.uint32(i + 1)
    return x0, x1


def _np_random_bits(k0, k1, n):
    o0, o1 = _np_threefry2x32(k0, k1, np.zeros(n, np.uint32),
                              np.arange(n, dtype=np.uint32))
    return (o0 ^ o1).astype(np.uint32)


@functools.lru_cache(maxsize=None)
def _neg_indices(seed, n, span):
    k0 = np.uint32((seed >> 32) & 0xFFFFFFFF)
    k1 = np.uint32(seed & 0xFFFFFFFF)
    o0, o1 = _np_threefry2x32(k0, k1, np.array([0, 0], np.uint32),
                              np.array([0, 1], np.uint32))
    higher = _np_random_bits(o0[0], o1[0], n)
    lower = _np_random_bits(o0[1], o1[1], n)
    span_u = np.uint32(span)
    with np.errstate(over='ignore'):
        multiplier = np.uint32(np.uint32(1 << 16) % span_u)
        multiplier = np.uint32((multiplier * multiplier) % span_u)
        off = ((higher % span_u) * multiplier + (lower % span_u)) % span_u
    return np.asarray(off, np.int32)


def _rsqrt(x):
    xi = plsc.bitcast(x, jnp.int32)
    y = plsc.bitcast(0x5F3759DF - (xi >> 1), jnp.float32)
    for _ in range(3):
        y = y * (1.5 - 0.5 * x * y * y)
    return y


def _build(P, D, NC, NS):
    NW = NC * NS
    BPW = P // NW
    NG = BPW // _L
    DH = D // 2
    UP = 8

    mesh = plsc.VectorSubcoreMesh(core_axis_name="c", subcore_axis_name="s")

    @functools.partial(
        pl.kernel,
        mesh=mesh,
        out_type=jax.ShapeDtypeStruct((NW, _L), jnp.float32),
        compiler_params=pltpu.CompilerParams(
            use_tc_tiling_on_sc=True, needs_layout_passes=False),
        scratch_types=[
            pltpu.VMEM((BPW,), jnp.int32),
            pltpu.VMEM((BPW,), jnp.int32),
            pltpu.VMEM((BPW,), jnp.int32),
            pltpu.VMEM((BPW, D), jnp.float32),
            pltpu.VMEM((BPW, D), jnp.float32),
            pltpu.VMEM((BPW, D), jnp.float32),
            pltpu.VMEM((8, BPW), jnp.float32),
            pltpu.VMEM((_L,), jnp.float32),
            pltpu.SemaphoreType.DMA,
            pltpu.SemaphoreType.DMA,
            pltpu.SemaphoreType.DMA,
        ],
    )
    def sc_loss(emb_hbm, kg_hbm, ii_hbm, jj_hbm, nn_hbm, out_hbm,
                idx_i, idx_j, idx_n, buf_a, buf_b, buf_c,
                dots, acc_v, sem_a, sem_b, sem_c):
        wid = lax.axis_index("s") * NC + lax.axis_index("c")
        base = wid * BPW
        ci = pltpu.async_copy(ii_hbm.at[pl.ds(base, BPW)], idx_i, sem_b)
        cj = pltpu.async_copy(jj_hbm.at[pl.ds(base, BPW)], idx_j, sem_b)
        cn = pltpu.async_copy(nn_hbm.at[pl.ds(base, BPW)], idx_n, sem_b)
        ci.wait()
        cj.wait()
        cn.wait()

        HB = BPW // 2
        hs0, hs1 = pl.ds(0, HB), pl.ds(HB, HB)
        i0, i1 = idx_i.at[hs0], idx_i.at[hs1]
        j0, j1 = idx_j.at[hs0], idx_j.at[hs1]
        n0, n1 = idx_n.at[hs0], idx_n.at[hs1]
        A0, A1 = buf_a.at[hs0, :], buf_a.at[hs1, :]
        B0, B1 = buf_b.at[hs0, :], buf_b.at[hs1, :]
        C0, C1 = buf_c.at[hs0, :], buf_c.at[hs1, :]

        c1 = pltpu.async_copy(emb_hbm.at[i0], A0, sem_a)
        c2 = pltpu.async_copy(emb_hbm.at[j0], B0, sem_a)
        c3 = pltpu.async_copy(emb_hbm.at[i1], A1, sem_a)
        c4 = pltpu.async_copy(emb_hbm.at[j1], B1, sem_a)
        c5 = pltpu.async_copy(emb_hbm.at[n0], C0, sem_a)
        c6 = pltpu.async_copy(kg_hbm.at[i0], C1, sem_a)

        def dot_pass(x_ref, y_ref, slots, g_lo, g_hi, x_off=0, y_off=0):
            def g_loop(g, _):
                rows = lax.iota(jnp.int32, _L) + g * _L
                rows_x = rows + x_off
                rows_y = rows + y_off
                z = jnp.zeros((_L,), jnp.float32)
                acc = (z, z, z, z, z, z)
                for h in (0, DH):
                    def d_body(c, carry):
                        colv, xy0, xy1, xx0, xx1, yy0, yy1 = carry
                        for u in range(UP):
                            cols = colv + h
                            x = plsc.load_gather(x_ref, [rows_x, cols])
                            y = plsc.load_gather(y_ref, [rows_y, cols])
                            if u % 2 == 0:
                                xy0 = xy0 + x * y
                                xx0 = xx0 + x * x
                                yy0 = yy0 + y * y
                            else:
                                xy1 = xy1 + x * y
                                xx1 = xx1 + x * x
                                yy1 = yy1 + y * y
                            colv = (colv + 1) & (DH - 1)
                        return colv, xy0, xy1, xx0, xx1, yy0, yy1

                    _, *acc = lax.fori_loop(
                        0, DH // UP, d_body,
                        (lax.iota(jnp.int32, _L), *acc))
                xy0, xy1, xx0, xx1, yy0, yy1 = acc
                vals = {'xy': xy0 + xy1, 'xx': xx0 + xx1, 'yy': yy0 + yy1}
                s = pl.ds(g * _L, _L)
                for row, kind in slots:
                    dots[row, s] = vals[kind]
                return 0
            lax.fori_loop(g_lo, g_hi, g_loop, 0)

        s1 = [(0, 'xy'), (1, 'xx'), (2, 'yy')]
        s2 = [(3, 'xy'), (4, 'yy')]
        s3 = [(5, 'xy'), (6, 'xx'), (7, 'yy')]
        NGH = NG // 2

        c1.wait()
        c2.wait()
        dot_pass(buf_a, buf_b, s1, 0, NGH)
        c7 = pltpu.async_copy(emb_hbm.at[n1], B0, sem_a)
        c3.wait()
        c4.wait()
        dot_pass(buf_a, buf_b, s1, NGH, NG)
        c8 = pltpu.async_copy(kg_hbm.at[i1], B1, sem_a)
        c5.wait()
        dot_pass(buf_a, buf_c, s2, 0, NGH)
        c9 = pltpu.async_copy(kg_hbm.at[j0], A0, sem_a)
        c6.wait()
        c7.wait()
        dot_pass(buf_a, buf_b, s2, NGH, NG, y_off=-HB)
        c10 = pltpu.async_copy(kg_hbm.at[j1], A1, sem_a)
        c8.wait()
        c9.wait()
        dot_pass(buf_c, buf_a, s3, 0, NGH, x_off=HB)
        c10.wait()
        dot_pass(buf_b, buf_a, s3, NGH, NG)

        def cos(ab, aa, bb):
            n2 = aa * bb
            nrm = n2 * _rsqrt(n2)
            return ab / jnp.maximum(nrm, 1e-8)

        def g_body(g, acc):
            s = pl.ds(g * _L, _L)
            sim_eb = cos(dots[0, s], dots[1, s], dots[2, s])
            neg_sim = cos(dots[3, s], dots[1, s], dots[4, s])
            sim_kg = cos(dots[5, s], dots[6, s], dots[7, s])
            per = (jnp.abs(sim_eb - sim_kg)
                   + 0.1 * jnp.maximum(0.0, 0.5 - neg_sim))
            iv, jv, nv = idx_i[s], idx_j[s], idx_n[s]
            mask = (nv != iv) & (nv != jv)
            return acc + jnp.where(mask, per, 0.0)
        acc = lax.fori_loop(0, NG, g_body, jnp.zeros((_L,), jnp.float32))

        acc_v[...] = acc
        pltpu.sync_copy(acc_v, out_hbm.at[wid])

    return sc_loss


def kernel(entity_embeddings, knowledge_embeddings, entity_pairs):
    P = entity_pairs.shape[0]
    N, D = entity_embeddings.shape
    info = plsc.get_sparse_core_info()
    NC, NS = info.num_cores, info.num_subcores

    ii = entity_pairs[:, 0].astype(jnp.int32)
    jj = entity_pairs[:, 1].astype(jnp.int32)
    nn = jnp.asarray(_neg_indices(42, P, N))

    partials = _build(P, D, NC, NS)(
        entity_embeddings, knowledge_embeddings, ii, jj, nn)
    return jnp.sum(partials) / max(P, 1)
